# baseline (speedup 1.0000x reference)
import functools

import jax
import jax.numpy as jnp
from jax import lax
from jax.experimental import pallas as pl
from jax.experimental.pallas import tpu as pltpu
from jax.experimental.pallas import tpu_sc as plsc

_BUF_ROWS = 24


def _sc_copy(seq, d, dtype):
    info = plsc.get_sparse_core_info()
    nc, ns = info.num_cores, info.num_subcores
    nw = nc * ns
    rows_per_w = seq // nw
    sizes = []
    left = rows_per_w
    while left > 0:
        c = min(_BUF_ROWS, left)
        sizes.append(c)
        left -= c
    offs = [sum(sizes[:i]) for i in range(len(sizes))]
    n_chunks = len(sizes)
    mesh = plsc.VectorSubcoreMesh(core_axis_name="c", subcore_axis_name="s")

    @functools.partial(
        pl.kernel,
        mesh=mesh,
        out_type=jax.ShapeDtypeStruct((seq, d), dtype),
        scratch_types=[
            pltpu.VMEM((2, _BUF_ROWS, d), dtype),
            pltpu.SemaphoreType.DMA,
            pltpu.SemaphoreType.DMA,
            pltpu.SemaphoreType.DMA,
            pltpu.SemaphoreType.DMA,
        ],
    )
    def k(emb_hbm, out_hbm, buf, si0, si1, so0, so1):
        in_sems = (si0, si1)
        out_sems = (so0, so1)
        wid = lax.axis_index("s") * nc + lax.axis_index("c")
        base = wid * rows_per_w

        def in_copy(i):
            return pltpu.make_async_copy(
                emb_hbm.at[pl.ds(base + offs[i], sizes[i])],
                buf.at[i % 2, pl.ds(0, sizes[i])], in_sems[i % 2])

        def out_copy(i):
            return pltpu.make_async_copy(
                buf.at[i % 2, pl.ds(0, sizes[i])],
                out_hbm.at[pl.ds(base + offs[i], sizes[i])],
                out_sems[i % 2])

        in_copy(0).start()
        for i in range(n_chunks):
            in_copy(i).wait()
            out_copy(i).start()
            if i + 1 < n_chunks:
                if i >= 1:
                    out_copy(i - 1).wait()
                in_copy(i + 1).start()
        out_copy(n_chunks - 2).wait()
        out_copy(n_chunks - 1).wait()

    return k


def kernel(x, emb):
    seq = x.shape[1]
    d = emb.shape[1]
    return _sc_copy(seq, d, emb.dtype)(emb)


# speedup vs baseline: 1.0066x; 1.0066x over previous
import functools

import jax
import jax.numpy as jnp
from jax import lax
from jax.experimental import pallas as pl
from jax.experimental.pallas import tpu as pltpu
from jax.experimental.pallas import tpu_sc as plsc

_CHUNK = 8
_NBUF = 6


def _sc_copy(seq, d, dtype):
    info = plsc.get_sparse_core_info()
    nc, ns = info.num_cores, info.num_subcores
    nw = nc * ns
    rows_per_w = seq // nw
    n_chunks = rows_per_w // _CHUNK
    mesh = plsc.VectorSubcoreMesh(core_axis_name="c", subcore_axis_name="s")

    @functools.partial(
        pl.kernel,
        mesh=mesh,
        out_type=jax.ShapeDtypeStruct((seq, d), dtype),
        scratch_types=(
            [pltpu.VMEM((_NBUF, _CHUNK, d), dtype)]
            + [pltpu.SemaphoreType.DMA] * (2 * _NBUF)
        ),
    )
    def k(emb_hbm, out_hbm, buf, *sems):
        in_sems = sems[:_NBUF]
        out_sems = sems[_NBUF:]
        wid = lax.axis_index("s") * nc + lax.axis_index("c")
        base = wid * rows_per_w

        def in_copy(i):
            return pltpu.make_async_copy(
                emb_hbm.at[pl.ds(base + i * _CHUNK, _CHUNK)],
                buf.at[i % _NBUF], in_sems[i % _NBUF])

        def out_copy(i):
            return pltpu.make_async_copy(
                buf.at[i % _NBUF],
                out_hbm.at[pl.ds(base + i * _CHUNK, _CHUNK)],
                out_sems[i % _NBUF])

        for i in range(_NBUF - 1):
            in_copy(i).start()
        for i in range(n_chunks):
            in_copy(i).wait()
            out_copy(i).start()
            j = i + _NBUF - 1
            if j < n_chunks:
                if j - _NBUF >= 0:
                    out_copy(j - _NBUF).wait()
                in_copy(j).start()
        for i in range(max(0, n_chunks - _NBUF), n_chunks):
            out_copy(i).wait()

    return k


def kernel(x, emb):
    seq = x.shape[1]
    d = emb.shape[1]
    return _sc_copy(seq, d, emb.dtype)(emb)
